# baseline (speedup 1.0000x reference)
import functools

import jax
import jax.numpy as jnp
from jax import lax
from jax.experimental import pallas as pl
from jax.experimental.pallas import tpu as pltpu
from jax.experimental.pallas import tpu_sc as plsc

B, N, D, E, K = 2, 2048, 768, 8, 2
NC, NS = 2, 16
NW = NC * NS
NIDX = 8192
BPW = NIDX // NW
CH = 64
D2 = D // 2

_mesh = plsc.VectorSubcoreMesh(core_axis_name="c", subcore_axis_name="s")


def _sc_gather(table, idx):
    @functools.partial(
        pl.kernel,
        out_type=jax.ShapeDtypeStruct((NIDX, D2), jnp.int32),
        mesh=_mesh,
        scratch_types=[
            pltpu.VMEM((BPW,), jnp.int32),
            pltpu.VMEM((CH, D2), jnp.int32),
            pltpu.VMEM((CH, D2), jnp.int32),
            pltpu.SemaphoreType.DMA,
            pltpu.SemaphoreType.DMA,
        ],
    )
    def k(table_hbm, idx_hbm, out_hbm, idx_v, buf0, buf1, sem0, sem1):
        wid = lax.axis_index("s") * NC + lax.axis_index("c")
        base = wid * BPW
        pltpu.sync_copy(idx_hbm.at[pl.ds(base, BPW)], idx_v)
        bufs = (buf0, buf1)
        sems = (sem0, sem1)
        nch = BPW // CH
        cps = []
        cps.append(
            pltpu.async_copy(table_hbm.at[idx_v.at[pl.ds(0, CH)]], bufs[0], sems[0])
        )
        for c in range(nch):
            if c + 1 < nch:
                cps.append(
                    pltpu.async_copy(
                        table_hbm.at[idx_v.at[pl.ds((c + 1) * CH, CH)]],
                        bufs[(c + 1) % 2],
                        sems[(c + 1) % 2],
                    )
                )
            cps[c].wait()
            pltpu.sync_copy(bufs[c % 2], out_hbm.at[pl.ds(base + c * CH, CH)])

    return k(table, idx)


def kernel(x, gate_w, w1, b1, w2, b2):
    xf = x.reshape(B * N, D).astype(jnp.bfloat16)
    xi = jax.lax.bitcast_convert_type(xf.reshape(B * N, D2, 2), jnp.int32)
    i = jnp.arange(NIDX, dtype=jnp.uint32)
    idx = ((i * jnp.uint32(2654435761)) % jnp.uint32(B * N)).astype(jnp.int32)
    g = _sc_gather(xi, idx)
    gb = jax.lax.bitcast_convert_type(g, jnp.bfloat16).reshape(NIDX, D)
    return gb[: B * N].astype(jnp.float32).reshape(B, N, D)


# speedup vs baseline: 1.2442x; 1.2442x over previous
import jax
import jax.numpy as jnp
from jax.experimental import pallas as pl
from jax.experimental.pallas import tpu as pltpu

B, N, D, E, K = 2, 2048, 768, 8, 2
T = B * N


TC = 512


def _moe_block(x_ref, gw_ref, w1_ref, b1_ref, w2_ref, b2_ref, o_ref, wt_ref):
    e = pl.program_id(0)
    inv_sqrt2 = 0.7071067811865476
    for c in range(T // TC):
        rows = pl.ds(c * TC, TC)
        xb = x_ref[rows, :]

        @pl.when(e == 0)
        def _router(xb=xb, rows=rows):
            logits = jnp.dot(xb, gw_ref[...], preferred_element_type=jnp.float32)
            probs = jax.nn.softmax(logits, axis=-1)
            e_ids = jax.lax.broadcasted_iota(jnp.int32, probs.shape, 1)
            i1 = jnp.argmax(probs, axis=-1)
            p1 = jnp.max(probs, axis=-1)
            sel1 = e_ids == i1[:, None]
            masked = jnp.where(sel1, -jnp.inf, probs)
            i2 = jnp.argmax(masked, axis=-1)
            p2 = jnp.max(masked, axis=-1)
            sel2 = e_ids == i2[:, None]
            wt = p1[:, None] * sel1.astype(jnp.float32) + p2[:, None] * sel2.astype(
                jnp.float32
            )
            wt_ref[rows, :] = wt
            o_ref[rows, :] = jnp.dot(wt, b2_ref[...], preferred_element_type=jnp.float32)

        h = jnp.dot(xb, w1_ref[0], preferred_element_type=jnp.float32)
        h = h + b1_ref[0]
        g = 0.5 * h * (1.0 + jax.lax.erf(h * inv_sqrt2))
        y = jnp.dot(g, w2_ref[0], preferred_element_type=jnp.float32)
        wt = wt_ref[rows, :]
        e_ids = jax.lax.broadcasted_iota(jnp.int32, wt.shape, 1)
        wcol = jnp.sum(
            jnp.where(e_ids == e, wt, 0.0), axis=-1, keepdims=True
        )
        o_ref[rows, :] += wcol * y


def kernel(x, gate_w, w1, b1, w2, b2):
    xf = x.reshape(T, D)
    b1 = b1.reshape(E, 1, D)
    out = pl.pallas_call(
        _moe_block,
        grid=(E,),
        in_specs=[
            pl.BlockSpec((T, D), lambda e: (0, 0)),
            pl.BlockSpec((D, E), lambda e: (0, 0)),
            pl.BlockSpec((1, D, D), lambda e: (e, 0, 0)),
            pl.BlockSpec((1, 1, D), lambda e: (e, 0, 0)),
            pl.BlockSpec((1, D, D), lambda e: (e, 0, 0)),
            pl.BlockSpec((E, D), lambda e: (0, 0)),
        ],
        out_specs=pl.BlockSpec((T, D), lambda e: (0, 0)),
        out_shape=jax.ShapeDtypeStruct((T, D), jnp.float32),
        scratch_shapes=[pltpu.VMEM((T, E), jnp.float32)],
        compiler_params=pltpu.CompilerParams(
            dimension_semantics=("arbitrary",),
        ),
    )(xf, gate_w, w1, b1, w2, b2)
    return out.reshape(B, N, D)


# speedup vs baseline: 1.7433x; 1.4012x over previous
import jax
import jax.numpy as jnp
from jax.experimental import pallas as pl
from jax.experimental.pallas import tpu as pltpu

B, N, D, E, K = 2, 2048, 768, 8, 2
TB = 1024


def _moe_block(x_ref, gw_ref, w1_ref, b1_ref, w2_ref, b2_ref, o_ref):
    xb = x_ref[...]
    logits = jnp.dot(xb, gw_ref[...], preferred_element_type=jnp.float32)
    probs = jax.nn.softmax(logits, axis=-1)
    e_ids = jax.lax.broadcasted_iota(jnp.int32, probs.shape, 1)
    i1 = jnp.argmax(probs, axis=-1)
    p1 = jnp.max(probs, axis=-1)
    sel1 = e_ids == i1[:, None]
    masked = jnp.where(sel1, -jnp.inf, probs)
    i2 = jnp.argmax(masked, axis=-1)
    p2 = jnp.max(masked, axis=-1)
    sel2 = e_ids == i2[:, None]
    wt = p1[:, None] * sel1.astype(jnp.float32) + p2[:, None] * sel2.astype(
        jnp.float32
    )

    acc = jnp.zeros((xb.shape[0], D), jnp.float32)
    inv_sqrt2 = 0.7071067811865476
    for e in range(E):
        h = jnp.dot(xb, w1_ref[e], preferred_element_type=jnp.float32)
        h = h + b1_ref[e][None, :]
        h = 0.5 * h * (1.0 + jax.lax.erf(h * inv_sqrt2))
        y = jnp.dot(h, w2_ref[e], preferred_element_type=jnp.float32)
        y = y + b2_ref[e][None, :]
        acc = acc + wt[:, e][:, None] * y
    o_ref[...] = acc


def kernel(x, gate_w, w1, b1, w2, b2):
    xf = x.reshape(B * N, D)
    grid = (B * N // TB,)
    out = pl.pallas_call(
        _moe_block,
        grid=grid,
        in_specs=[
            pl.BlockSpec((TB, D), lambda i: (i, 0)),
            pl.BlockSpec((D, E), lambda i: (0, 0)),
            pl.BlockSpec((E, D, D), lambda i: (0, 0, 0)),
            pl.BlockSpec((E, D), lambda i: (0, 0)),
            pl.BlockSpec((E, D, D), lambda i: (0, 0, 0)),
            pl.BlockSpec((E, D), lambda i: (0, 0)),
        ],
        out_specs=pl.BlockSpec((TB, D), lambda i: (i, 0)),
        out_shape=jax.ShapeDtypeStruct((B * N, D), jnp.float32),
        compiler_params=pltpu.CompilerParams(
            dimension_semantics=("arbitrary",),
        ),
    )(xf, gate_w, w1, b1, w2, b2)
    return out.reshape(B, N, D)
